# baseline (speedup 1.0000x reference)
import functools

import jax
import jax.numpy as jnp
from jax import lax
from jax.experimental import pallas as pl
from jax.experimental.pallas import tpu as pltpu
from jax.experimental.pallas import tpu_sc as plsc

B = 32
N = 64
D = 256
P = 4096

NC = 2
NS = 16
NW = NC * NS

CPQ = 64
NCHUNK = P // CPQ
NBUF = 2

_MESH = plsc.VectorSubcoreMesh(core_axis_name="c", subcore_axis_name="s")


@functools.partial(
    pl.kernel,
    mesh=_MESH,
    compiler_params=pltpu.CompilerParams(needs_layout_passes=False),
    out_type=jax.ShapeDtypeStruct((B, P, 2 * D), jnp.float32),
    scratch_types=[
        pltpu.VMEM((N * D,), jnp.float32),
        pltpu.VMEM((NCHUNK, CPQ), jnp.int32),
        pltpu.VMEM((NCHUNK, CPQ), jnp.int32),
        pltpu.VMEM((CPQ, 2 * D), jnp.float32),
        pltpu.VMEM((CPQ, 2 * D), jnp.float32),
        pltpu.SemaphoreType.DMA,
        pltpu.SemaphoreType.DMA,
        pltpu.SemaphoreType.DMA,
        pltpu.SemaphoreType.DMA,
    ],
)
def _pair_gather(table_hbm, xf_hbm, i_hbm, j_hbm, out_hbm, xs, iv, jv,
                 buf0, buf1, g0, g1, s0, s1):
    w = lax.axis_index("s") * NC + lax.axis_index("c")
    base = w * N
    bufs = (buf0, buf1)
    gsem = (g0, g1)
    ssem = (s0, s1)

    pltpu.sync_copy(xf_hbm.at[w], xs)
    pltpu.sync_copy(i_hbm, iv)
    pltpu.sync_copy(j_hbm, jv)

    def prep_body(ci, carry):
        for t in range(CPQ // 16):
            sl = pl.ds(t * 16, 16)
            iv[ci, sl] = iv[ci, sl] + base
            jv[ci, sl] = jv[ci, sl] + base
        return carry

    lax.fori_loop(0, NCHUNK, prep_body, 0)

    def fire_gather(ci, b):
        pltpu.async_copy(
            table_hbm.at[iv.at[ci]], bufs[b].at[:, pl.ds(0, D)], gsem[b])
        pltpu.async_copy(
            table_hbm.at[jv.at[ci]], bufs[b].at[:, pl.ds(D, D)], gsem[b])

    def wait_gather(ci, b):
        pltpu.make_async_copy(
            table_hbm.at[iv.at[ci]], bufs[b].at[:, pl.ds(0, D)],
            gsem[b]).wait()
        pltpu.make_async_copy(
            table_hbm.at[jv.at[ci]], bufs[b].at[:, pl.ds(D, D)],
            gsem[b]).wait()

    def fire_scatter(ci, b):
        pltpu.async_copy(
            bufs[b], out_hbm.at[w, pl.ds(ci * CPQ, CPQ)], ssem[b])

    def wait_scatter(ci, b):
        pltpu.make_async_copy(
            bufs[b], out_hbm.at[w, pl.ds(ci * CPQ, CPQ)], ssem[b]).wait()

    UNR = 2

    def fill(ci, buf):
        iota0 = lax.iota(jnp.int32, 16)
        cvec = iota0 - base * D

        def pair_body(q, carry):
            srcs = []
            for k in range(UNR):
                p = UNR * q + k
                pv = jnp.full((16,), p, dtype=jnp.int32)
                civ = jnp.full((16,), ci, dtype=jnp.int32)
                r1 = plsc.load_gather(iv, [civ, pv])
                r2 = plsc.load_gather(jv, [civ, pv])
                srcs.append((p, r1 * D + cvec, r2 * D + cvec))
            for t in range(D // 16):
                off = 16 * t
                for p, g1, g2 in srcs:
                    buf[p, pl.ds(off, 16)] = plsc.load_gather(
                        xs, [g1 + off])
                    buf[p, pl.ds(D + off, 16)] = plsc.load_gather(
                        xs, [g2 + off])
            return carry

        lax.fori_loop(0, CPQ // UNR, pair_body, 0)

    for b in range(NBUF):
        fire_gather(b, b)

    def chunk_body(g, carry):
        for b in range(NBUF):
            ci = NBUF * g + b
            if b == 1:
                odd = (g % 2) == 1

                @pl.when(odd)
                def _():
                    fill(ci, bufs[1])

                @pl.when(jnp.logical_not(odd))
                def _():
                    wait_gather(ci, 1)
            else:
                wait_gather(ci, 0)
            fire_scatter(ci, b)
            pb = (b - 1) % NBUF

            @pl.when(jnp.logical_and(
                jnp.logical_and(ci >= 1, ci + NBUF - 1 < NCHUNK),
                (ci + 1) % 4 != 3))
            def _():
                wait_scatter(ci - 1, pb)
                fire_gather(ci + NBUF - 1, pb)

            @pl.when(jnp.logical_and(
                jnp.logical_and(ci >= 1, ci + NBUF - 1 < NCHUNK),
                (ci + 1) % 4 == 3))
            def _():
                wait_scatter(ci - 1, pb)
        return carry

    lax.fori_loop(0, NCHUNK // NBUF, chunk_body, 0)

    for k in range(NBUF):
        ci = NCHUNK - NBUF + k
        wait_scatter(ci, ci % NBUF)


def kernel(x, i, j):
    table = x.reshape(B * N, D)
    xf = x.reshape(B, N * D)
    i2 = i.reshape(NCHUNK, CPQ)
    j2 = j.reshape(NCHUNK, CPQ)
    return _pair_gather(table, xf, i2, j2)


# speedup vs baseline: 1.2512x; 1.2512x over previous
import functools

import jax
import jax.numpy as jnp
from jax import lax
from jax.experimental import pallas as pl
from jax.experimental.pallas import tpu as pltpu
from jax.experimental.pallas import tpu_sc as plsc

B = 32
N = 64
D = 256
P = 4096

NC = 2
NS = 16
NW = NC * NS

C = 128
NCHUNK = P // C

_MESH = plsc.VectorSubcoreMesh(core_axis_name="c", subcore_axis_name="s")


@functools.partial(
    pl.kernel,
    mesh=_MESH,
    out_type=jax.ShapeDtypeStruct((B, P, 2 * D), jnp.float32),
    scratch_types=[
        pltpu.VMEM((NCHUNK, C), jnp.int32),
        pltpu.VMEM((C, D), jnp.float32),
        pltpu.VMEM((C, D), jnp.float32),
        pltpu.SemaphoreType.DMA,
        pltpu.SemaphoreType.DMA,
    ],
)
def _pair_gather(table_hbm, i_hbm, j_hbm, out_hbm, idx_v, rows0, rows1,
                 gsem0, gsem1):
    w = lax.axis_index("s") * NC + lax.axis_index("c")
    base = w * N
    bufs = ((rows0, gsem0), (rows1, gsem1))

    for half, sel_hbm in ((0, i_hbm), (1, j_hbm)):
        pltpu.sync_copy(sel_hbm, idx_v)

        def prep_body(ci, carry):
            for t in range(C // 16):
                sl = pl.ds(t * 16, 16)
                idx_v[ci, sl] = idx_v[ci, sl] + base
            return carry

        lax.fori_loop(0, NCHUNK, prep_body, 0)

        for b, (rows, gsem) in enumerate(bufs):
            pltpu.async_copy(table_hbm.at[idx_v.at[b]], rows, gsem)

        def chunk_body(g, carry, half=half):
            for b, (rows, gsem) in enumerate(bufs):
                ci = 2 * g + b
                pltpu.make_async_copy(
                    table_hbm.at[idx_v.at[ci]], rows, gsem).wait()
                pltpu.sync_copy(
                    rows,
                    out_hbm.at[w, pl.ds(ci * C, C), pl.ds(half * D, D)],
                )
                nci = ci + 2

                @pl.when(nci < NCHUNK)
                def _():
                    pltpu.async_copy(table_hbm.at[idx_v.at[nci]], rows, gsem)
            return carry

        lax.fori_loop(0, NCHUNK // 2, chunk_body, 0)


def kernel(x, i, j):
    table = x.reshape(B * N, D)
    i2 = i.reshape(NCHUNK, C)
    j2 = j.reshape(NCHUNK, C)
    return _pair_gather(table, i2, j2)
